# baseline (speedup 1.0000x reference)
import jax
import jax.numpy as jnp
from jax.experimental import pallas as pl


from jax.experimental.pallas import tpu as pltpu

S_BLK = 512


def _bcast_kernel(table_ref, out_ref):
    out_ref[...] = jnp.broadcast_to(table_ref[...][None], out_ref.shape)


def kernel(inputs, table):
    batch, seq = inputs.shape
    n_rows, d_model = table.shape
    grid = (seq // S_BLK,)
    return pl.pallas_call(
        _bcast_kernel,
        grid=grid,
        in_specs=[
            pl.BlockSpec((S_BLK, d_model), lambda i: (i, 0)),
        ],
        out_specs=pl.BlockSpec((batch, S_BLK, d_model), lambda i: (0, i, 0)),
        out_shape=jax.ShapeDtypeStruct((batch, seq, d_model), table.dtype),
        compiler_params=pltpu.CompilerParams(
            dimension_semantics=("parallel",),
        ),
    )(table)


# speedup vs baseline: 1.0388x; 1.0388x over previous
import jax
import jax.numpy as jnp
from jax.experimental import pallas as pl


from jax.experimental.pallas import tpu as pltpu

S_BLK = 1024


def _bcast_kernel(table_ref, out_ref):
    out_ref[...] = jnp.broadcast_to(table_ref[...][None], out_ref.shape)


def kernel(inputs, table):
    batch, seq = inputs.shape
    n_rows, d_model = table.shape
    grid = (seq // S_BLK,)
    return pl.pallas_call(
        _bcast_kernel,
        grid=grid,
        in_specs=[
            pl.BlockSpec((S_BLK, d_model), lambda i: (i, 0)),
        ],
        out_specs=pl.BlockSpec((batch, S_BLK, d_model), lambda i: (0, i, 0)),
        out_shape=jax.ShapeDtypeStruct((batch, seq, d_model), table.dtype),
        compiler_params=pltpu.CompilerParams(
            dimension_semantics=("parallel",),
        ),
    )(table)
